# Initial kernel scaffold; baseline (speedup 1.0000x reference)
import jax
import jax.numpy as jnp
from jax.experimental import pallas as pl


def kernel(pos, x, edge_index, edge_vec, batch, w1_0, w2_0, W_0, w1_1, w2_1, W_1, w1_2, w2_2, W_2, w1_3, w2_3, W_3):
    raise NotImplementedError("write your pallas kernel here")


# speedup vs baseline: 2.9891x; 2.9891x over previous
import functools

import numpy as np
import jax
import jax.numpy as jnp
from jax import lax
from jax.experimental import pallas as pl
from jax.experimental.pallas import tpu as pltpu
from jax.experimental.pallas import tpu_sc as plsc

N_NODES = 10000
N_EDGES = 160000
NB = 10
MAX_R = 2.0

NC = 2
NS = 16
LANES = 16

EDGES_PER_SUB = N_EDGES // NS
CHUNK = 400
NCHUNK = EDGES_PER_SUB // CHUNK
BATCHES = CHUNK // LANES
ROWS_PER_SUB = N_NODES // NS


_EB = 2000


def _edge_feat_body(ev_ref, w10, w20, w11, w21, w12, w22, w13, w23,
                    a0, a1, a2, a3):
    ev = ev_ref[...]
    vx = ev[:, 0:1]
    vy = ev[:, 1:2]
    vz = ev[:, 2:3]
    n = jnp.sqrt(vx * vx + vy * vy + vz * vz)
    inv = 1.0 / jnp.maximum(n, 1e-9)
    ux = vx * inv
    uy = vy * inv
    uz = vz * inv
    c1 = float(np.sqrt(3.0))
    c2 = float(np.sqrt(15.0))
    c3 = float(np.sqrt(5.0) / 2.0)
    sh = jnp.concatenate(
        [jnp.ones_like(ux), c1 * uy, c1 * uz, c1 * ux,
         c2 * ux * uy, c2 * uy * uz, c3 * (3.0 * uz * uz - 1.0),
         c2 * ux * uz, (c2 / 2.0) * (ux * ux - uy * uy)], axis=1)

    centers = np.linspace(0.0, MAX_R, NB + 2)[1:-1]
    step = float(centers[1] - centers[0])
    diff = jnp.concatenate([(n - float(ck)) / step for ck in centers], axis=1)
    inside = jnp.abs(diff) < 1.0
    denom = jnp.where(inside, 1.0 - diff * diff, 1.0)
    amp = float(1.14136 * np.exp(2.0))
    emb = jnp.where(inside, amp * jnp.exp(-1.0 / denom), 0.0) * float(np.sqrt(NB))

    for (w1, w2, aref) in ((w10, w20, a0), (w11, w21, a1),
                           (w12, w22, a2), (w13, w23, a3)):
        hmid = jax.nn.silu(jnp.dot(emb, w1[...],
                                   preferred_element_type=jnp.float32))
        wk = jnp.dot(hmid, w2[...], preferred_element_type=jnp.float32)
        aref[...] = sh * wk


def _edge_features(edge_vec, ws):
    w_specs = []
    for w in ws:
        w_specs.append(pl.BlockSpec(w.shape, lambda e: (0, 0)))
    out_shape = [jax.ShapeDtypeStruct((N_EDGES, 9), jnp.float32)] * 4
    out_specs = [pl.BlockSpec((_EB, 9), lambda e: (e, 0))] * 4
    return pl.pallas_call(
        _edge_feat_body,
        grid=(N_EDGES // _EB,),
        in_specs=[pl.BlockSpec((_EB, 3), lambda e: (e, 0))] + w_specs,
        out_specs=out_specs,
        out_shape=out_shape,
    )(edge_vec, *ws)


_MB = 2000


def _mm0_body(x_ref, w_ref, g_ref):
    g_ref[...] = jnp.dot(x_ref[...], w_ref[...],
                         preferred_element_type=jnp.float32
                         ).astype(jnp.bfloat16)


def _mm0(x, wt):
    din, cout = wt.shape[1], wt.shape[2]
    return pl.pallas_call(
        _mm0_body,
        grid=(NC, N_NODES // _MB),
        in_specs=[pl.BlockSpec((_MB, din), lambda c, m: (m, 0)),
                  pl.BlockSpec((None, din, cout), lambda c, m: (c, 0, 0))],
        out_specs=pl.BlockSpec((None, _MB, cout), lambda c, m: (c, m, 0)),
        out_shape=jax.ShapeDtypeStruct((NC, N_NODES, cout), jnp.bfloat16),
    )(x, wt)


def _mm_silu_body(s_ref, w_ref, g_ref):
    dh = s_ref.shape[-1]
    a0 = jax.nn.silu(s_ref[0])
    a1 = jax.nn.silu(s_ref[1])
    g_ref[...] = (jnp.dot(a0, w_ref[0:dh], preferred_element_type=jnp.float32)
                  + jnp.dot(a1, w_ref[dh:2 * dh],
                            preferred_element_type=jnp.float32)
                  ).astype(jnp.bfloat16)


def _mm_silu(s, wt):
    dh_in = s.shape[-1]
    din, cout = wt.shape[1], wt.shape[2]
    return pl.pallas_call(
        _mm_silu_body,
        grid=(NC, N_NODES // _MB),
        in_specs=[pl.BlockSpec((2, _MB, dh_in), lambda c, m: (0, m, 0)),
                  pl.BlockSpec((None, din, cout), lambda c, m: (c, 0, 0))],
        out_specs=pl.BlockSpec((None, _MB, cout), lambda c, m: (c, m, 0)),
        out_shape=jax.ShapeDtypeStruct((NC, N_NODES, cout), jnp.bfloat16),
    )(s, wt)


_PB = 2000


def _pool_body(s_ref, o_ref):
    m = pl.program_id(0)

    @pl.when(m == 0)
    def _():
        o_ref[...] = jnp.zeros_like(o_ref)

    o_ref[...] += jnp.sum(s_ref[...], axis=1) * 0.01


def _pool(s):
    dh = s.shape[-1]
    return pl.pallas_call(
        _pool_body,
        grid=(N_NODES // _PB,),
        in_specs=[pl.BlockSpec((2, _PB, dh), lambda m: (0, m, 0))],
        out_specs=pl.BlockSpec((2, dh), lambda m: (0, 0)),
        out_shape=jax.ShapeDtypeStruct((2, dh), jnp.float32),
    )(s)


@functools.lru_cache(maxsize=None)
def _make_sc_layer(dh):
    groups = dh // 32
    j_chunks = dh // LANES
    row_w = 9 * dh
    mesh = plsc.VectorSubcoreMesh(core_axis_name="c", subcore_axis_name="s",
                                  num_cores=NC, num_subcores=NS)

    @functools.partial(
        pl.kernel,
        out_type=jax.ShapeDtypeStruct((NC * N_NODES, dh), jnp.float32),
        mesh=mesh,
        scratch_types=[
            pltpu.VMEM_SHARED((N_NODES, dh), jnp.float32),
            pltpu.VMEM((CHUNK,), jnp.int32),
            pltpu.VMEM((CHUNK,), jnp.int32),
            pltpu.VMEM((CHUNK * 9,), jnp.float32),
            pltpu.VMEM((LANES, row_w), jnp.bfloat16),
            pltpu.VMEM((LANES, row_w), jnp.bfloat16),
            pltpu.VMEM((LANES, dh), jnp.float32),
            pltpu.SemaphoreType.DMA,
            pltpu.SemaphoreType.DMA,
        ],
        compiler_params=pltpu.CompilerParams(use_tc_tiling_on_sc=False,
                                             needs_layout_passes=False),
    )
    def sc_layer(g_hbm, src_hbm, dst_hbm, attr_hbm, zrow_hbm, out_hbm,
                 acc, src_v, dst_v, attr_v, rows_a, rows_b, msg_v,
                 sem_a, sem_b):
        c = lax.axis_index("c")
        s = lax.axis_index("s")
        node_off = c * N_NODES

        pltpu.sync_copy(zrow_hbm, acc.at[pl.ds(s * ROWS_PER_SUB, ROWS_PER_SUB)])
        plsc.subcore_barrier()

        def gather(b, rows, sem):
            vec = src_v[pl.ds(b * LANES, LANES)] + node_off
            pltpu.async_copy(g_hbm.at[vec], rows, sem)

        def gwait(rows, sem):
            pltpu.make_async_copy(g_hbm.at[pl.ds(0, LANES)], rows, sem).wait()

        def compute(b, rows):
            for i in range(LANES):
                accs = [None] * j_chunks
                for k in range(9):
                    aidx = jnp.full((LANES,), b * (LANES * 9) + i * 9 + k,
                                    jnp.int32)
                    a = plsc.load_gather(attr_v, [aidx])
                    for g in range(groups):
                        v = rows[i, pl.ds(k * dh + g * 32, 32)]
                        lo, hi = plsc.unpack(
                            v, format=plsc.PackFormat.INTERLEAVED)
                        if k == 0:
                            accs[2 * g] = a * lo
                            accs[2 * g + 1] = a * hi
                        else:
                            accs[2 * g] = accs[2 * g] + a * lo
                            accs[2 * g + 1] = accs[2 * g + 1] + a * hi
                for j in range(j_chunks):
                    msg_v[i, pl.ds(j * LANES, LANES)] = accs[j]
            dvec = dst_v[pl.ds(b * LANES, LANES)]
            pltpu.sync_copy(msg_v, acc.at[dvec], add=True)

        def chunk_body(cc, carry):
            base = s * EDGES_PER_SUB + cc * CHUNK
            pltpu.sync_copy(src_hbm.at[pl.ds(base, CHUNK)], src_v)
            pltpu.sync_copy(dst_hbm.at[pl.ds(base, CHUNK)], dst_v)
            pltpu.sync_copy(attr_hbm.at[pl.ds(base * 9, CHUNK * 9)], attr_v)

            gather(0, rows_a, sem_a)

            def pair_body(t, carry2):
                b0 = 2 * t
                gather(b0 + 1, rows_b, sem_b)
                gwait(rows_a, sem_a)
                compute(b0, rows_a)
                gather(b0 + 2, rows_a, sem_a)
                gwait(rows_b, sem_b)
                compute(b0 + 1, rows_b)
                return carry2

            lax.fori_loop(0, (BATCHES - 1) // 2, pair_body, 0)
            gwait(rows_a, sem_a)
            compute(BATCHES - 1, rows_a)
            return carry

        lax.fori_loop(0, NCHUNK, chunk_body, 0)
        plsc.subcore_barrier()

        sl = pl.ds(s * ROWS_PER_SUB, ROWS_PER_SUB)
        pltpu.sync_copy(acc.at[sl],
                        out_hbm.at[pl.ds(node_off + s * ROWS_PER_SUB,
                                         ROWS_PER_SUB)])

    return sc_layer


_PERM32 = np.arange(32).reshape(2, 16).T.ravel()


def _interleave_cols(w):
    cols = w.shape[1]
    perm = (np.arange(0, cols, 32)[:, None] + _PERM32[None, :]).ravel()
    return w[:, perm]


def _prep_weight(W, din, dout, dh_out, dh_in_pad=None):
    dh = dout // 2
    wt = (W * 0.25).reshape(9, din, dout).transpose(1, 0, 2)
    halves = []
    for c in (0, 1):
        wh = wt[:, :, c * dh:(c + 1) * dh]
        wh = jnp.pad(wh, ((0, 0), (0, 0), (0, dh_out - dh)))
        wh = _interleave_cols(wh.reshape(din, 9 * dh_out))
        halves.append(wh)
    w2 = jnp.stack(halves)
    if dh_in_pad is not None:
        dh_in = din // 2
        w2 = w2.reshape(2, 2, dh_in, 9 * dh_out)
        w2 = jnp.pad(w2, ((0, 0), (0, 0), (0, dh_in_pad - dh_in), (0, 0)))
        w2 = w2.reshape(2, 2 * dh_in_pad, 9 * dh_out)
    return w2


def kernel(pos, x, edge_index, edge_vec, batch,
           w1_0, w2_0, W_0, w1_1, w2_1, W_1,
           w1_2, w2_2, W_2, w1_3, w2_3, W_3):
    del pos, batch
    src = edge_index[0]
    dst = edge_index[1]

    a0, a1, a2, a3 = _edge_features(
        edge_vec, (w1_0, w2_0, w1_1, w2_1, w1_2, w2_2, w1_3, w2_3))

    dhp = 160
    wt0 = _prep_weight(W_0, 128, 288, dhp)
    wt1 = _prep_weight(W_1, 288, 288, dhp, dh_in_pad=dhp)
    wt2 = _prep_weight(W_2, 288, 288, dhp, dh_in_pad=dhp)
    wt3 = _prep_weight(W_3, 288, 128, 64, dh_in_pad=dhp)

    zr160 = jnp.zeros((ROWS_PER_SUB, dhp), jnp.float32)
    zr64 = jnp.zeros((ROWS_PER_SUB, 64), jnp.float32)

    sc160 = _make_sc_layer(dhp)
    sc64 = _make_sc_layer(64)

    g = _mm0(x, wt0).reshape(NC * N_NODES, 9 * dhp)
    s1 = sc160(g, src, dst, a0.reshape(-1), zr160)

    g = _mm_silu(s1.reshape(NC, N_NODES, dhp), wt1).reshape(NC * N_NODES,
                                                            9 * dhp)
    s2 = sc160(g, src, dst, a1.reshape(-1), zr160)

    g = _mm_silu(s2.reshape(NC, N_NODES, dhp), wt2).reshape(NC * N_NODES,
                                                            9 * dhp)
    s3 = sc160(g, src, dst, a2.reshape(-1), zr160)

    g = _mm_silu(s3.reshape(NC, N_NODES, dhp), wt3).reshape(NC * N_NODES,
                                                            9 * 64)
    s4 = sc64(g, src, dst, a3.reshape(-1), zr64)

    res = _pool(s4.reshape(NC, N_NODES, 64))
    return res.reshape(1, 128)


# speedup vs baseline: 4.6674x; 1.5614x over previous
import functools

import numpy as np
import jax
import jax.numpy as jnp
from jax import lax
from jax.experimental import pallas as pl
from jax.experimental.pallas import tpu as pltpu
from jax.experimental.pallas import tpu_sc as plsc

N_NODES = 10000
N_EDGES = 160000
NB = 10
MAX_R = 2.0

NC = 2
NS = 16
LANES = 16

EDGES_PER_SUB = N_EDGES // NS
CHUNK = 400
NCHUNK = EDGES_PER_SUB // CHUNK
BATCHES = CHUNK // LANES
ROWS_PER_SUB = N_NODES // NS


_EB = 2000


def _edge_feat_body(ev_ref, w10, w20, w11, w21, w12, w22, w13, w23,
                    a0, a1, a2, a3):
    ev = ev_ref[...]
    vx = ev[:, 0:1]
    vy = ev[:, 1:2]
    vz = ev[:, 2:3]
    n = jnp.sqrt(vx * vx + vy * vy + vz * vz)
    inv = 1.0 / jnp.maximum(n, 1e-9)
    ux = vx * inv
    uy = vy * inv
    uz = vz * inv
    c1 = float(np.sqrt(3.0))
    c2 = float(np.sqrt(15.0))
    c3 = float(np.sqrt(5.0) / 2.0)
    sh = jnp.concatenate(
        [jnp.ones_like(ux), c1 * uy, c1 * uz, c1 * ux,
         c2 * ux * uy, c2 * uy * uz, c3 * (3.0 * uz * uz - 1.0),
         c2 * ux * uz, (c2 / 2.0) * (ux * ux - uy * uy)], axis=1)

    centers = np.linspace(0.0, MAX_R, NB + 2)[1:-1]
    step = float(centers[1] - centers[0])
    diff = jnp.concatenate([(n - float(ck)) / step for ck in centers], axis=1)
    inside = jnp.abs(diff) < 1.0
    denom = jnp.where(inside, 1.0 - diff * diff, 1.0)
    amp = float(1.14136 * np.exp(2.0))
    emb = jnp.where(inside, amp * jnp.exp(-1.0 / denom), 0.0) * float(np.sqrt(NB))

    for (w1, w2, aref) in ((w10, w20, a0), (w11, w21, a1),
                           (w12, w22, a2), (w13, w23, a3)):
        hmid = jax.nn.silu(jnp.dot(emb, w1[...],
                                   preferred_element_type=jnp.float32))
        wk = jnp.dot(hmid, w2[...], preferred_element_type=jnp.float32)
        aref[...] = sh * wk


def _edge_features(edge_vec, ws):
    w_specs = []
    for w in ws:
        w_specs.append(pl.BlockSpec(w.shape, lambda e: (0, 0)))
    out_shape = [jax.ShapeDtypeStruct((N_EDGES, 9), jnp.float32)] * 4
    out_specs = [pl.BlockSpec((_EB, 9), lambda e: (e, 0))] * 4
    return pl.pallas_call(
        _edge_feat_body,
        grid=(N_EDGES // _EB,),
        in_specs=[pl.BlockSpec((_EB, 3), lambda e: (e, 0))] + w_specs,
        out_specs=out_specs,
        out_shape=out_shape,
    )(edge_vec, *ws)


_MB = 2000


def _mm0_body(x_ref, w_ref, g_ref):
    g_ref[...] = jnp.dot(x_ref[...], w_ref[...],
                         preferred_element_type=jnp.float32)


def _mm0(x, wt):
    din, cout = wt.shape[1], wt.shape[2]
    return pl.pallas_call(
        _mm0_body,
        grid=(NC, N_NODES // _MB),
        in_specs=[pl.BlockSpec((_MB, din), lambda c, m: (m, 0)),
                  pl.BlockSpec((None, din, cout), lambda c, m: (c, 0, 0))],
        out_specs=pl.BlockSpec((None, _MB, cout), lambda c, m: (c, m, 0)),
        out_shape=jax.ShapeDtypeStruct((NC, N_NODES, cout), jnp.float32),
    )(x, wt)


def _mm_silu_body(s_ref, w_ref, g_ref):
    dh = s_ref.shape[-1]
    a0 = jax.nn.silu(s_ref[0])
    a1 = jax.nn.silu(s_ref[1])
    g_ref[...] = (jnp.dot(a0, w_ref[0:dh], preferred_element_type=jnp.float32)
                  + jnp.dot(a1, w_ref[dh:2 * dh],
                            preferred_element_type=jnp.float32))


def _mm_silu(s, wt):
    dh_in = s.shape[-1]
    din, cout = wt.shape[1], wt.shape[2]
    return pl.pallas_call(
        _mm_silu_body,
        grid=(NC, N_NODES // _MB),
        in_specs=[pl.BlockSpec((2, _MB, dh_in), lambda c, m: (0, m, 0)),
                  pl.BlockSpec((None, din, cout), lambda c, m: (c, 0, 0))],
        out_specs=pl.BlockSpec((None, _MB, cout), lambda c, m: (c, m, 0)),
        out_shape=jax.ShapeDtypeStruct((NC, N_NODES, cout), jnp.float32),
    )(s, wt)


_PB = 2000


def _pool_body(s_ref, o_ref):
    m = pl.program_id(0)

    @pl.when(m == 0)
    def _():
        o_ref[...] = jnp.zeros_like(o_ref)

    o_ref[...] += jnp.sum(s_ref[...], axis=1) * 0.01


def _pool(s):
    dh = s.shape[-1]
    return pl.pallas_call(
        _pool_body,
        grid=(N_NODES // _PB,),
        in_specs=[pl.BlockSpec((2, _PB, dh), lambda m: (0, m, 0))],
        out_specs=pl.BlockSpec((2, dh), lambda m: (0, 0)),
        out_shape=jax.ShapeDtypeStruct((2, dh), jnp.float32),
    )(s)


@functools.lru_cache(maxsize=None)
def _make_sc_layer(dh):
    j_chunks = dh // LANES
    row_w = 9 * dh
    half = LANES // 2
    mesh = plsc.VectorSubcoreMesh(core_axis_name="c", subcore_axis_name="s",
                                  num_cores=NC, num_subcores=NS)

    @functools.partial(
        pl.kernel,
        out_type=jax.ShapeDtypeStruct((NC * N_NODES, dh), jnp.float32),
        mesh=mesh,
        scratch_types=[
            pltpu.VMEM_SHARED((N_NODES, dh), jnp.float32),
            pltpu.VMEM((CHUNK,), jnp.int32),
            pltpu.VMEM((CHUNK,), jnp.int32),
            pltpu.VMEM((CHUNK * 9,), jnp.float32),
            pltpu.VMEM((half, row_w), jnp.float32),
            pltpu.VMEM((half, row_w), jnp.float32),
            pltpu.VMEM((LANES, dh), jnp.float32),
            pltpu.SemaphoreType.DMA,
            pltpu.SemaphoreType.DMA,
        ],
        compiler_params=pltpu.CompilerParams(use_tc_tiling_on_sc=False,
                                             needs_layout_passes=False),
    )
    def sc_layer(g_hbm, src2_hbm, dst_hbm, attr_hbm, zrow_hbm, out_hbm,
                 acc, src_v, dst_v, attr_v, rows_a, rows_b, msg_v,
                 sem_a, sem_b):
        c = lax.axis_index("c")
        s = lax.axis_index("s")
        node_off = c * N_NODES

        pltpu.sync_copy(zrow_hbm, acc.at[pl.ds(s * ROWS_PER_SUB, ROWS_PER_SUB)])
        plsc.subcore_barrier()

        def issue(h, rows, sem):
            pltpu.async_copy(g_hbm.at[src_v.at[pl.ds(h * half, half)]],
                             rows, sem)

        def gwait(rows, sem):
            pltpu.make_async_copy(g_hbm.at[pl.ds(0, half)], rows, sem).wait()

        def compute_half(b, hb, rows):
            for ii in range(half):
                i = hb * half + ii
                accs = [None] * j_chunks
                for k in range(9):
                    aidx = jnp.full((LANES,), b * (LANES * 9) + i * 9 + k,
                                    jnp.int32)
                    a = plsc.load_gather(attr_v, [aidx])
                    for j in range(j_chunks):
                        r = rows[ii, pl.ds((k * j_chunks + j) * LANES, LANES)]
                        if k == 0:
                            accs[j] = a * r
                        else:
                            accs[j] = accs[j] + a * r
                for j in range(j_chunks):
                    msg_v[i, pl.ds(j * LANES, LANES)] = accs[j]

        def scatter(b):
            dvec = dst_v[pl.ds(b * LANES, LANES)]
            pltpu.sync_copy(msg_v, acc.at[dvec], add=True)

        def chunk_body(cc, carry):
            base = s * EDGES_PER_SUB + cc * CHUNK
            pltpu.sync_copy(src2_hbm.at[c].at[pl.ds(base, CHUNK)], src_v)
            pltpu.sync_copy(dst_hbm.at[pl.ds(base, CHUNK)], dst_v)
            pltpu.sync_copy(attr_hbm.at[pl.ds(base * 9, CHUNK * 9)], attr_v)

            issue(0, rows_a, sem_a)

            def batch_body(b, carry2):
                issue(2 * b + 1, rows_b, sem_b)
                gwait(rows_a, sem_a)
                compute_half(b, 0, rows_a)
                issue(2 * b + 2, rows_a, sem_a)
                gwait(rows_b, sem_b)
                compute_half(b, 1, rows_b)
                scatter(b)
                return carry2

            lax.fori_loop(0, BATCHES - 1, batch_body, 0)
            b = BATCHES - 1
            issue(2 * b + 1, rows_b, sem_b)
            gwait(rows_a, sem_a)
            compute_half(b, 0, rows_a)
            gwait(rows_b, sem_b)
            compute_half(b, 1, rows_b)
            scatter(b)
            return carry

        lax.fori_loop(0, NCHUNK, chunk_body, 0)
        plsc.subcore_barrier()

        sl = pl.ds(s * ROWS_PER_SUB, ROWS_PER_SUB)
        pltpu.sync_copy(acc.at[sl],
                        out_hbm.at[pl.ds(node_off + s * ROWS_PER_SUB,
                                         ROWS_PER_SUB)])

    return sc_layer


def _prep_weight(W, din, dout):
    dh = dout // 2
    wt = (W * 0.25).reshape(9, din, dout).transpose(1, 0, 2)
    lo = wt[:, :, :dh].reshape(din, 9 * dh)
    hi = wt[:, :, dh:].reshape(din, 9 * dh)
    return jnp.stack([lo, hi])


def kernel(pos, x, edge_index, edge_vec, batch,
           w1_0, w2_0, W_0, w1_1, w2_1, W_1,
           w1_2, w2_2, W_2, w1_3, w2_3, W_3):
    del pos, batch
    src = edge_index[0]
    dst = edge_index[1]
    src2 = jnp.stack([src, src + N_NODES])

    a0, a1, a2, a3 = _edge_features(
        edge_vec, (w1_0, w2_0, w1_1, w2_1, w1_2, w2_2, w1_3, w2_3))

    wt0 = _prep_weight(W_0, 128, 288)
    wt1 = _prep_weight(W_1, 288, 288)
    wt2 = _prep_weight(W_2, 288, 288)
    wt3 = _prep_weight(W_3, 288, 128)

    zr144 = jnp.zeros((ROWS_PER_SUB, 144), jnp.float32)
    zr64 = jnp.zeros((ROWS_PER_SUB, 64), jnp.float32)

    sc144 = _make_sc_layer(144)
    sc64 = _make_sc_layer(64)

    g = _mm0(x, wt0).reshape(NC * N_NODES, 9 * 144)
    s1 = sc144(g, src2, dst, a0.reshape(-1), zr144)

    g = _mm_silu(s1.reshape(NC, N_NODES, 144), wt1).reshape(NC * N_NODES,
                                                            9 * 144)
    s2 = sc144(g, src2, dst, a1.reshape(-1), zr144)

    g = _mm_silu(s2.reshape(NC, N_NODES, 144), wt2).reshape(NC * N_NODES,
                                                            9 * 144)
    s3 = sc144(g, src2, dst, a2.reshape(-1), zr144)

    g = _mm_silu(s3.reshape(NC, N_NODES, 144), wt3).reshape(NC * N_NODES,
                                                            9 * 64)
    s4 = sc64(g, src2, dst, a3.reshape(-1), zr64)

    res = _pool(s4.reshape(NC, N_NODES, 64))
    return res.reshape(1, 128)


# speedup vs baseline: 4.6772x; 1.0021x over previous
import functools

import numpy as np
import jax
import jax.numpy as jnp
from jax import lax
from jax.experimental import pallas as pl
from jax.experimental.pallas import tpu as pltpu
from jax.experimental.pallas import tpu_sc as plsc

N_NODES = 10000
N_EDGES = 160000
NB = 10
MAX_R = 2.0

NC = 2
NS = 16
LANES = 16

EDGES_PER_SUB = N_EDGES // NS
CHUNK = 400
NCHUNK = EDGES_PER_SUB // CHUNK
BATCHES = CHUNK // LANES
ROWS_PER_SUB = N_NODES // NS


_EB = 2000


def _edge_feat_body(ev_ref, w10, w20, w11, w21, w12, w22, w13, w23,
                    a0, a1, a2, a3):
    ev = ev_ref[...]
    vx = ev[:, 0:1]
    vy = ev[:, 1:2]
    vz = ev[:, 2:3]
    n = jnp.sqrt(vx * vx + vy * vy + vz * vz)
    inv = 1.0 / jnp.maximum(n, 1e-9)
    ux = vx * inv
    uy = vy * inv
    uz = vz * inv
    c1 = float(np.sqrt(3.0))
    c2 = float(np.sqrt(15.0))
    c3 = float(np.sqrt(5.0) / 2.0)
    sh = jnp.concatenate(
        [jnp.ones_like(ux), c1 * uy, c1 * uz, c1 * ux,
         c2 * ux * uy, c2 * uy * uz, c3 * (3.0 * uz * uz - 1.0),
         c2 * ux * uz, (c2 / 2.0) * (ux * ux - uy * uy)], axis=1)

    centers = np.linspace(0.0, MAX_R, NB + 2)[1:-1]
    step = float(centers[1] - centers[0])
    diff = jnp.concatenate([(n - float(ck)) / step for ck in centers], axis=1)
    inside = jnp.abs(diff) < 1.0
    denom = jnp.where(inside, 1.0 - diff * diff, 1.0)
    amp = float(1.14136 * np.exp(2.0))
    emb = jnp.where(inside, amp * jnp.exp(-1.0 / denom), 0.0) * float(np.sqrt(NB))

    for (w1, w2, aref) in ((w10, w20, a0), (w11, w21, a1),
                           (w12, w22, a2), (w13, w23, a3)):
        hmid = jax.nn.silu(jnp.dot(emb, w1[...],
                                   preferred_element_type=jnp.float32))
        wk = jnp.dot(hmid, w2[...], preferred_element_type=jnp.float32)
        aref[...] = sh * wk


def _edge_features(edge_vec, ws):
    w_specs = []
    for w in ws:
        w_specs.append(pl.BlockSpec(w.shape, lambda e: (0, 0)))
    out_shape = [jax.ShapeDtypeStruct((N_EDGES, 9), jnp.float32)] * 4
    out_specs = [pl.BlockSpec((_EB, 9), lambda e: (e, 0))] * 4
    return pl.pallas_call(
        _edge_feat_body,
        grid=(N_EDGES // _EB,),
        in_specs=[pl.BlockSpec((_EB, 3), lambda e: (e, 0))] + w_specs,
        out_specs=out_specs,
        out_shape=out_shape,
    )(edge_vec, *ws)


_MB = 2000


def _mm0_body(x_ref, w_ref, g_ref):
    g_ref[...] = jnp.dot(x_ref[...].astype(jnp.bfloat16), w_ref[...],
                         preferred_element_type=jnp.float32)


def _mm0(x, wt):
    din, cout = wt.shape[1], wt.shape[2]
    return pl.pallas_call(
        _mm0_body,
        grid=(NC, N_NODES // _MB),
        in_specs=[pl.BlockSpec((_MB, din), lambda c, m: (m, 0)),
                  pl.BlockSpec((None, din, cout), lambda c, m: (c, 0, 0))],
        out_specs=pl.BlockSpec((None, _MB, cout), lambda c, m: (c, m, 0)),
        out_shape=jax.ShapeDtypeStruct((NC, N_NODES, cout), jnp.float32),
    )(x, wt)


def _mm_silu_body(s_ref, w_ref, g_ref):
    dh = s_ref.shape[-1]
    a0 = jax.nn.silu(s_ref[0]).astype(jnp.bfloat16)
    a1 = jax.nn.silu(s_ref[1]).astype(jnp.bfloat16)
    g_ref[...] = (jnp.dot(a0, w_ref[0:dh], preferred_element_type=jnp.float32)
                  + jnp.dot(a1, w_ref[dh:2 * dh],
                            preferred_element_type=jnp.float32))


def _mm_silu(s, wt):
    dh_in = s.shape[-1]
    din, cout = wt.shape[1], wt.shape[2]
    return pl.pallas_call(
        _mm_silu_body,
        grid=(NC, N_NODES // _MB),
        in_specs=[pl.BlockSpec((2, _MB, dh_in), lambda c, m: (0, m, 0)),
                  pl.BlockSpec((None, din, cout), lambda c, m: (c, 0, 0))],
        out_specs=pl.BlockSpec((None, _MB, cout), lambda c, m: (c, m, 0)),
        out_shape=jax.ShapeDtypeStruct((NC, N_NODES, cout), jnp.float32),
    )(s, wt)


_PB = 2000


def _pool_body(s_ref, o_ref):
    m = pl.program_id(0)

    @pl.when(m == 0)
    def _():
        o_ref[...] = jnp.zeros_like(o_ref)

    o_ref[...] += jnp.sum(s_ref[...], axis=1) * 0.01


def _pool(s):
    dh = s.shape[-1]
    return pl.pallas_call(
        _pool_body,
        grid=(N_NODES // _PB,),
        in_specs=[pl.BlockSpec((2, _PB, dh), lambda m: (0, m, 0))],
        out_specs=pl.BlockSpec((2, dh), lambda m: (0, 0)),
        out_shape=jax.ShapeDtypeStruct((2, dh), jnp.float32),
    )(s)


@functools.lru_cache(maxsize=None)
def _make_sc_layer(dh):
    j_chunks = dh // LANES
    row_w = 9 * dh
    half = LANES // 2
    mesh = plsc.VectorSubcoreMesh(core_axis_name="c", subcore_axis_name="s",
                                  num_cores=NC, num_subcores=NS)

    @functools.partial(
        pl.kernel,
        out_type=jax.ShapeDtypeStruct((NC * N_NODES, dh), jnp.float32),
        mesh=mesh,
        scratch_types=[
            pltpu.VMEM_SHARED((N_NODES, dh), jnp.float32),
            pltpu.VMEM((CHUNK,), jnp.int32),
            pltpu.VMEM((CHUNK,), jnp.int32),
            pltpu.VMEM((CHUNK * 9,), jnp.float32),
            pltpu.VMEM((half, row_w), jnp.float32),
            pltpu.VMEM((half, row_w), jnp.float32),
            pltpu.VMEM((LANES, dh), jnp.float32),
            pltpu.SemaphoreType.DMA,
            pltpu.SemaphoreType.DMA,
        ],
        compiler_params=pltpu.CompilerParams(use_tc_tiling_on_sc=False,
                                             needs_layout_passes=False),
    )
    def sc_layer(g_hbm, src2_hbm, dst_hbm, attr_hbm, zrow_hbm, out_hbm,
                 acc, src_v, dst_v, attr_v, rows_a, rows_b, msg_v,
                 sem_a, sem_b):
        c = lax.axis_index("c")
        s = lax.axis_index("s")
        node_off = c * N_NODES

        pltpu.sync_copy(zrow_hbm, acc.at[pl.ds(s * ROWS_PER_SUB, ROWS_PER_SUB)])
        plsc.subcore_barrier()

        def issue(h, rows, sem):
            pltpu.async_copy(g_hbm.at[src_v.at[pl.ds(h * half, half)]],
                             rows, sem)

        def gwait(rows, sem):
            pltpu.make_async_copy(g_hbm.at[pl.ds(0, half)], rows, sem).wait()

        def compute_half(b, hb, rows):
            for ii in range(half):
                i = hb * half + ii
                accs = [None] * j_chunks
                for k in range(9):
                    aidx = jnp.full((LANES,), b * (LANES * 9) + i * 9 + k,
                                    jnp.int32)
                    a = plsc.load_gather(attr_v, [aidx])
                    for j in range(j_chunks):
                        r = rows[ii, pl.ds((k * j_chunks + j) * LANES, LANES)]
                        if k == 0:
                            accs[j] = a * r
                        else:
                            accs[j] = accs[j] + a * r
                for j in range(j_chunks):
                    msg_v[i, pl.ds(j * LANES, LANES)] = accs[j]

        def scatter(b):
            dvec = dst_v[pl.ds(b * LANES, LANES)]
            pltpu.sync_copy(msg_v, acc.at[dvec], add=True)

        def chunk_body(cc, carry):
            base = s * EDGES_PER_SUB + cc * CHUNK
            pltpu.sync_copy(src2_hbm.at[c].at[pl.ds(base, CHUNK)], src_v)
            pltpu.sync_copy(dst_hbm.at[pl.ds(base, CHUNK)], dst_v)
            pltpu.sync_copy(attr_hbm.at[pl.ds(base * 9, CHUNK * 9)], attr_v)

            issue(0, rows_a, sem_a)

            def batch_body(b, carry2):
                issue(2 * b + 1, rows_b, sem_b)
                gwait(rows_a, sem_a)
                compute_half(b, 0, rows_a)
                issue(2 * b + 2, rows_a, sem_a)
                gwait(rows_b, sem_b)
                compute_half(b, 1, rows_b)
                scatter(b)
                return carry2

            lax.fori_loop(0, BATCHES - 1, batch_body, 0)
            b = BATCHES - 1
            issue(2 * b + 1, rows_b, sem_b)
            gwait(rows_a, sem_a)
            compute_half(b, 0, rows_a)
            gwait(rows_b, sem_b)
            compute_half(b, 1, rows_b)
            scatter(b)
            return carry

        lax.fori_loop(0, NCHUNK, chunk_body, 0)
        plsc.subcore_barrier()

        sl = pl.ds(s * ROWS_PER_SUB, ROWS_PER_SUB)
        pltpu.sync_copy(acc.at[sl],
                        out_hbm.at[pl.ds(node_off + s * ROWS_PER_SUB,
                                         ROWS_PER_SUB)])

    return sc_layer


def _prep_weight(W, din, dout):
    dh = dout // 2
    wt = (W * 0.25).reshape(9, din, dout).transpose(1, 0, 2)
    lo = wt[:, :, :dh].reshape(din, 9 * dh)
    hi = wt[:, :, dh:].reshape(din, 9 * dh)
    return jnp.stack([lo, hi]).astype(jnp.bfloat16)


def kernel(pos, x, edge_index, edge_vec, batch,
           w1_0, w2_0, W_0, w1_1, w2_1, W_1,
           w1_2, w2_2, W_2, w1_3, w2_3, W_3):
    del pos, batch
    src = edge_index[0]
    dst = edge_index[1]
    src2 = jnp.stack([src, src + N_NODES])

    a0, a1, a2, a3 = _edge_features(
        edge_vec, (w1_0, w2_0, w1_1, w2_1, w1_2, w2_2, w1_3, w2_3))

    wt0 = _prep_weight(W_0, 128, 288)
    wt1 = _prep_weight(W_1, 288, 288)
    wt2 = _prep_weight(W_2, 288, 288)
    wt3 = _prep_weight(W_3, 288, 128)

    zr144 = jnp.zeros((ROWS_PER_SUB, 144), jnp.float32)
    zr64 = jnp.zeros((ROWS_PER_SUB, 64), jnp.float32)

    sc144 = _make_sc_layer(144)
    sc64 = _make_sc_layer(64)

    g = _mm0(x, wt0).reshape(NC * N_NODES, 9 * 144)
    s1 = sc144(g, src2, dst, a0.reshape(-1), zr144)

    g = _mm_silu(s1.reshape(NC, N_NODES, 144), wt1).reshape(NC * N_NODES,
                                                            9 * 144)
    s2 = sc144(g, src2, dst, a1.reshape(-1), zr144)

    g = _mm_silu(s2.reshape(NC, N_NODES, 144), wt2).reshape(NC * N_NODES,
                                                            9 * 144)
    s3 = sc144(g, src2, dst, a2.reshape(-1), zr144)

    g = _mm_silu(s3.reshape(NC, N_NODES, 144), wt3).reshape(NC * N_NODES,
                                                            9 * 64)
    s4 = sc64(g, src2, dst, a3.reshape(-1), zr64)

    res = _pool(s4.reshape(NC, N_NODES, 64))
    return res.reshape(1, 128)
